# Initial kernel scaffold; baseline (speedup 1.0000x reference)
import jax
import jax.numpy as jnp
from jax.experimental import pallas as pl


def kernel(x, pos_table):
    raise NotImplementedError("write your pallas kernel here")


# speedup vs baseline: 1.6360x; 1.6360x over previous
import jax
import jax.numpy as jnp
from jax.experimental import pallas as pl

BATCH = 4
MAXLEN = 2048
EMBED_DIM = 1024
T_BLK = 512


def _add_kernel(x_ref, pos_ref, out_ref):
    out_ref[...] = x_ref[...] + pos_ref[...]


def kernel(x, pos_table):
    x = jnp.reshape(x, (BATCH, MAXLEN, EMBED_DIM))
    grid = (BATCH, MAXLEN // T_BLK)
    return pl.pallas_call(
        _add_kernel,
        grid=grid,
        in_specs=[
            pl.BlockSpec((1, T_BLK, EMBED_DIM), lambda b, t: (b, t, 0)),
            pl.BlockSpec((T_BLK, EMBED_DIM), lambda b, t: (t, 0)),
        ],
        out_specs=pl.BlockSpec((1, T_BLK, EMBED_DIM), lambda b, t: (b, t, 0)),
        out_shape=jax.ShapeDtypeStruct((BATCH, MAXLEN, EMBED_DIM), x.dtype),
    )(x, pos_table)
